# baseline (speedup 1.0000x reference)
import functools

import jax
import jax.numpy as jnp
from jax import lax
from jax.experimental import pallas as pl
from jax.experimental.pallas import tpu as pltpu
from jax.experimental.pallas import tpu_sc as plsc

N = 10000
NP = 10240
E = 320000
NC = 2
NS = 16
EB = 64
KA = 136
KB = 184
KMAX = max(KA, KB)
NBLK = (KA + KB) // 2
FB = NS * (KA + KB)
FBP = FB + KMAX
EPAD = FB * EB
ROWS_T = NP // NS
RB = 3
DEG_Q = 8

_MESH = plsc.VectorSubcoreMesh(core_axis_name="c", subcore_axis_name="s")
_SC_PARAMS = pltpu.CompilerParams(use_tc_tiling_on_sc=False)


@functools.partial(
    pl.kernel,
    out_type=jax.ShapeDtypeStruct((NC, 2, NP), jnp.float32),
    mesh=_MESH,
    scratch_types=[
        pltpu.VMEM((NBLK, EB), jnp.int32),
        pltpu.VMEM((NBLK, EB), jnp.int32),
        pltpu.VMEM((EB,), jnp.float32),
        pltpu.VMEM((ROWS_T,), jnp.float32),
        pltpu.VMEM_SHARED((NP,), jnp.float32),
        pltpu.VMEM_SHARED((NP,), jnp.float32),
        pltpu.SemaphoreType.DMA,
    ],
    compiler_params=_SC_PARAMS,
)
def _deg_kernel(src_hbm, dst_hbm, out_hbm,
                src_v, dst_v, ones_v, zbuf_v, ho_sh, hi_sh, sem):
    c = lax.axis_index("c")
    s = lax.axis_index("s")
    w = c * NS + s
    base = s * ROWS_T

    def zinit(i, carry):
        zbuf_v[pl.ds(i * 16, 16)] = jnp.zeros((16,), jnp.float32)
        return carry

    lax.fori_loop(0, ROWS_T // 16, zinit, 0)

    def oinit(i, carry):
        ones_v[pl.ds(i * 16, 16)] = jnp.ones((16,), jnp.float32)
        return carry

    lax.fori_loop(0, EB // 16, oinit, 0)

    pltpu.sync_copy(zbuf_v, ho_sh.at[pl.ds(base, ROWS_T)])
    pltpu.sync_copy(zbuf_v, hi_sh.at[pl.ds(base, ROWS_T)])
    pltpu.sync_copy(src_hbm.at[pl.ds(w * NBLK, NBLK)], src_v)
    pltpu.sync_copy(dst_hbm.at[pl.ds(w * NBLK, NBLK)], dst_v)
    plsc.subcore_barrier()

    def fire(t):
        j = lax.rem(t, NBLK)

        @pl.when(t < NBLK)
        def _():
            pltpu.async_copy(ones_v, ho_sh.at[src_v.at[j]], sem, add=True)

        @pl.when(t >= NBLK)
        def _():
            pltpu.async_copy(ones_v, hi_sh.at[dst_v.at[j]], sem, add=True)

    def drain_one():
        pltpu.make_async_copy(ones_v, ho_sh.at[src_v.at[0]], sem).wait()

    def prol(t, carry):
        fire(t)
        return carry

    lax.fori_loop(0, DEG_Q, prol, 0)

    def steady(t, carry):
        fire(t + DEG_Q)
        drain_one()
        return carry

    lax.fori_loop(0, 2 * NBLK - DEG_Q, steady, 0)

    def epi(t, carry):
        drain_one()
        return carry

    lax.fori_loop(0, DEG_Q, epi, 0)

    plsc.subcore_barrier()
    pltpu.sync_copy(ho_sh.at[pl.ds(base, ROWS_T)],
                    out_hbm.at[c, 0, pl.ds(base, ROWS_T)])
    pltpu.sync_copy(hi_sh.at[pl.ds(base, ROWS_T)],
                    out_hbm.at[c, 1, pl.ds(base, ROWS_T)])


def _make_agg_kernel(D):
    @functools.partial(
        pl.kernel,
        out_type=jax.ShapeDtypeStruct((NC, NP, D), jnp.float32),
        mesh=_MESH,
        scratch_types=[
            pltpu.VMEM((KMAX, EB), jnp.int32),
            pltpu.VMEM((KMAX, EB), jnp.int32),
            pltpu.VMEM((RB, EB, D), jnp.float32),
            pltpu.VMEM_SHARED((NP, D), jnp.float32),
            pltpu.SemaphoreType.DMA((RB,)),
            pltpu.SemaphoreType.DMA((RB,)),
        ],
        compiler_params=_SC_PARAMS,
    )
    def agg(h_hbm, src_hbm, dst_hbm, out_hbm,
            src_v, dst_v, rows_v, agg_sh, gsem, ssem):
        c = lax.axis_index("c")
        s = lax.axis_index("s")
        base = s * ROWS_T
        nb = jnp.where(c == 0, KA, KB)
        base_blk = jnp.where(c == 0, s * KA, NS * KA + s * KB)

        def zinit(i, carry):
            rows_v[0, lax.div(i, D // 16), pl.ds(lax.rem(i, D // 16) * 16, 16)] = (
                jnp.zeros((16,), jnp.float32))
            return carry

        lax.fori_loop(0, EB * D // 16, zinit, 0)

        def zcopy(k, carry):
            pltpu.sync_copy(rows_v.at[0],
                            agg_sh.at[pl.ds(base + k * EB, EB)])
            return carry

        lax.fori_loop(0, ROWS_T // EB, zcopy, 0)

        pltpu.sync_copy(src_hbm.at[pl.ds(base_blk, KMAX)], src_v)
        pltpu.sync_copy(dst_hbm.at[pl.ds(base_blk, KMAX)], dst_v)
        plsc.subcore_barrier()

        def gather(j):
            b = lax.rem(j, RB)
            pltpu.async_copy(h_hbm.at[src_v.at[j]], rows_v.at[b], gsem.at[b])

        def wait_gather(j):
            b = lax.rem(j, RB)
            pltpu.make_async_copy(h_hbm.at[src_v.at[j]], rows_v.at[b],
                                  gsem.at[b]).wait()

        def scatter(j):
            b = lax.rem(j, RB)
            pltpu.async_copy(rows_v.at[b], agg_sh.at[dst_v.at[j]],
                             ssem.at[b], add=True)

        def wait_scatter(j):
            b = lax.rem(j, RB)
            pltpu.make_async_copy(rows_v.at[b], agg_sh.at[dst_v.at[j]],
                                  ssem.at[b]).wait()

        def prol(j, carry):
            gather(j)
            return carry

        lax.fori_loop(0, RB, prol, 0)

        def steady(j, carry):
            wait_gather(j)
            scatter(j)

            @pl.when(j >= 1)
            def _():
                wait_scatter(j - 1)

                @pl.when(j - 1 + RB < nb)
                def _():
                    gather(j - 1 + RB)

            return carry

        lax.fori_loop(0, nb, steady, 0)
        wait_scatter(nb - 1)

        plsc.subcore_barrier()
        pltpu.sync_copy(agg_sh.at[pl.ds(base, ROWS_T)],
                        out_hbm.at[c, pl.ds(base, ROWS_T)])

    return agg


_agg128 = _make_agg_kernel(128)
_agg64 = _make_agg_kernel(64)


def _norm(da, db):
    deg = da + db
    return jnp.where(deg > 0, lax.rsqrt(jnp.maximum(deg, 1.0)), 0.0)


def _mm1_body(x_ref, w_ref, doa_ref, dob_ref, o_ref):
    ns = _norm(doa_ref[...], dob_ref[...])
    o_ref[...] = jnp.dot(x_ref[...], w_ref[...],
                         preferred_element_type=jnp.float32) * ns


def _l2_body(p_ref, dia_ref, dib_ref, doa_ref, dob_ref,
             b1_ref, w2_ref, o_ref):
    nd = _norm(dia_ref[...], dib_ref[...])
    x = (p_ref[0] + p_ref[1]) * nd + b1_ref[...]
    x = jnp.maximum(x, 0.0)
    ns = _norm(doa_ref[...], dob_ref[...])
    o_ref[...] = jnp.dot(x, w2_ref[...],
                         preferred_element_type=jnp.float32) * ns


def _fin_body(p_ref, dia_ref, dib_ref, b2_ref, o_ref):
    nd = _norm(dia_ref[...], dib_ref[...])
    y = (p_ref[0] + p_ref[1]) * nd + b2_ref[...]
    m = jnp.max(y, axis=1, keepdims=True)
    e = jnp.exp(y - m)
    o_ref[...] = (y - m) - jnp.log(jnp.sum(e, axis=1, keepdims=True))


_RBK = 2560
_FBK = 2000


def _vec_spec(rb):
    return pl.BlockSpec((rb, 1), lambda i: (i, 0))


def _pair_spec(rb, d):
    return pl.BlockSpec((2, rb, d), lambda i: (0, i, 0))


def _full_spec(shape):
    return pl.BlockSpec(shape, lambda i: tuple(0 for _ in shape))


def _tc_mm1(x, w1, doa, dob):
    return pl.pallas_call(
        _mm1_body,
        grid=(NP // _RBK,),
        in_specs=[
            pl.BlockSpec((_RBK, 128), lambda i: (i, 0)),
            _full_spec((128, 128)),
            _vec_spec(_RBK), _vec_spec(_RBK),
        ],
        out_specs=pl.BlockSpec((_RBK, 128), lambda i: (i, 0)),
        out_shape=jax.ShapeDtypeStruct((NP, 128), jnp.float32),
    )(x, w1, doa, dob)


def _tc_l2(p, dia, dib, doa, dob, b1, w2):
    return pl.pallas_call(
        _l2_body,
        grid=(NP // _RBK,),
        in_specs=[
            _pair_spec(_RBK, 128),
            _vec_spec(_RBK), _vec_spec(_RBK), _vec_spec(_RBK), _vec_spec(_RBK),
            _full_spec((1, 128)),
            _full_spec((128, 64)),
        ],
        out_specs=pl.BlockSpec((_RBK, 64), lambda i: (i, 0)),
        out_shape=jax.ShapeDtypeStruct((NP, 64), jnp.float32),
    )(p, dia, dib, doa, dob, b1, w2)


def _tc_fin(p, dia, dib, b2):
    return pl.pallas_call(
        _fin_body,
        grid=(N // _FBK,),
        in_specs=[
            _pair_spec(_FBK, 64),
            _vec_spec(_FBK), _vec_spec(_FBK),
            _full_spec((1, 64)),
        ],
        out_specs=pl.BlockSpec((_FBK, 64), lambda i: (i, 0)),
        out_shape=jax.ShapeDtypeStruct((N, 64), jnp.float32),
    )(p, dia, dib, b2)


def kernel(features, edge_index, W1, b1, W2, b2):
    src = edge_index[0].astype(jnp.int32)
    dst = edge_index[1].astype(jnp.int32)
    pad = FBP * EB - E
    src_b = jnp.concatenate(
        [src, jnp.full((pad,), N, jnp.int32)]).reshape(FBP, EB)
    dst_b = jnp.concatenate(
        [dst, jnp.full((pad,), N, jnp.int32)]).reshape(FBP, EB)

    degp = _deg_kernel(src_b, dst_b)
    doa = degp[0, 0][:, None]
    dob = degp[1, 0][:, None]
    dia = degp[0, 1][:, None]
    dib = degp[1, 1][:, None]

    xpad = jnp.zeros((NP, 128), jnp.float32).at[:N].set(features)
    h1 = _tc_mm1(xpad, W1, doa, dob)
    p1 = _agg128(h1, src_b, dst_b)
    h2 = _tc_l2(p1, dia, dib, doa, dob,
                b1.reshape(1, 128), W2)
    p2 = _agg64(h2, src_b, dst_b)
    return _tc_fin(p2, dia, dib, b2.reshape(1, 64))


# speedup vs baseline: 1.0188x; 1.0188x over previous
import functools

import jax
import jax.numpy as jnp
from jax import lax
from jax.experimental import pallas as pl
from jax.experimental.pallas import tpu as pltpu
from jax.experimental.pallas import tpu_sc as plsc

N = 10000
NP = 10240
E = 320000
NC = 2
NS = 16
EB = 64
KA = 160
KB = 160
KMAX = max(KA, KB)
NBLK = (KA + KB) // 2
FB = NS * (KA + KB)
FBP = FB + KMAX
EPAD = FB * EB
ROWS_T = NP // NS
RB = 3
DEG_Q = 8

_MESH = plsc.VectorSubcoreMesh(core_axis_name="c", subcore_axis_name="s")
_SC_PARAMS = pltpu.CompilerParams(use_tc_tiling_on_sc=False)


@functools.partial(
    pl.kernel,
    out_type=jax.ShapeDtypeStruct((NC, 2, NP), jnp.float32),
    mesh=_MESH,
    scratch_types=[
        pltpu.VMEM((NBLK, EB), jnp.int32),
        pltpu.VMEM((NBLK, EB), jnp.int32),
        pltpu.VMEM((EB,), jnp.float32),
        pltpu.VMEM((ROWS_T,), jnp.float32),
        pltpu.VMEM_SHARED((NP,), jnp.float32),
        pltpu.VMEM_SHARED((NP,), jnp.float32),
        pltpu.SemaphoreType.DMA,
    ],
    compiler_params=_SC_PARAMS,
)
def _deg_kernel(src_hbm, dst_hbm, out_hbm,
                src_v, dst_v, ones_v, zbuf_v, ho_sh, hi_sh, sem):
    c = lax.axis_index("c")
    s = lax.axis_index("s")
    w = c * NS + s
    base = s * ROWS_T

    def zinit(i, carry):
        zbuf_v[pl.ds(i * 16, 16)] = jnp.zeros((16,), jnp.float32)
        return carry

    lax.fori_loop(0, ROWS_T // 16, zinit, 0)

    def oinit(i, carry):
        ones_v[pl.ds(i * 16, 16)] = jnp.ones((16,), jnp.float32)
        return carry

    lax.fori_loop(0, EB // 16, oinit, 0)

    pltpu.sync_copy(zbuf_v, ho_sh.at[pl.ds(base, ROWS_T)])
    pltpu.sync_copy(zbuf_v, hi_sh.at[pl.ds(base, ROWS_T)])
    pltpu.sync_copy(src_hbm.at[pl.ds(w * NBLK, NBLK)], src_v)
    pltpu.sync_copy(dst_hbm.at[pl.ds(w * NBLK, NBLK)], dst_v)
    plsc.subcore_barrier()

    def fire(t):
        j = lax.rem(t, NBLK)

        @pl.when(t < NBLK)
        def _():
            pltpu.async_copy(ones_v, ho_sh.at[src_v.at[j]], sem, add=True)

        @pl.when(t >= NBLK)
        def _():
            pltpu.async_copy(ones_v, hi_sh.at[dst_v.at[j]], sem, add=True)

    def drain_one():
        pltpu.make_async_copy(ones_v, ho_sh.at[src_v.at[0]], sem).wait()

    def prol(t, carry):
        fire(t)
        return carry

    lax.fori_loop(0, DEG_Q, prol, 0)

    def steady(t, carry):
        fire(t + DEG_Q)
        drain_one()
        return carry

    lax.fori_loop(0, 2 * NBLK - DEG_Q, steady, 0)

    def epi(t, carry):
        drain_one()
        return carry

    lax.fori_loop(0, DEG_Q, epi, 0)

    plsc.subcore_barrier()
    pltpu.sync_copy(ho_sh.at[pl.ds(base, ROWS_T)],
                    out_hbm.at[c, 0, pl.ds(base, ROWS_T)])
    pltpu.sync_copy(hi_sh.at[pl.ds(base, ROWS_T)],
                    out_hbm.at[c, 1, pl.ds(base, ROWS_T)])


def _make_agg_kernel(D):
    @functools.partial(
        pl.kernel,
        out_type=jax.ShapeDtypeStruct((NC, NP, D), jnp.float32),
        mesh=_MESH,
        scratch_types=[
            pltpu.VMEM((KMAX, EB), jnp.int32),
            pltpu.VMEM((KMAX, EB), jnp.int32),
            pltpu.VMEM((RB, EB, D), jnp.float32),
            pltpu.VMEM_SHARED((NP, D), jnp.float32),
            pltpu.SemaphoreType.DMA((RB,)),
            pltpu.SemaphoreType.DMA((RB,)),
        ],
        compiler_params=_SC_PARAMS,
    )
    def agg(h_hbm, src_hbm, dst_hbm, out_hbm,
            src_v, dst_v, rows_v, agg_sh, gsem, ssem):
        c = lax.axis_index("c")
        s = lax.axis_index("s")
        base = s * ROWS_T
        nb = NBLK
        base_blk = (c * NS + s) * NBLK

        def zinit(i, carry):
            rows_v[0, lax.div(i, D // 16), pl.ds(lax.rem(i, D // 16) * 16, 16)] = (
                jnp.zeros((16,), jnp.float32))
            return carry

        lax.fori_loop(0, EB * D // 16, zinit, 0)

        def zcopy(k, carry):
            pltpu.sync_copy(rows_v.at[0],
                            agg_sh.at[pl.ds(base + k * EB, EB)])
            return carry

        lax.fori_loop(0, ROWS_T // EB, zcopy, 0)

        pltpu.sync_copy(src_hbm.at[pl.ds(base_blk, KMAX)], src_v)
        pltpu.sync_copy(dst_hbm.at[pl.ds(base_blk, KMAX)], dst_v)
        plsc.subcore_barrier()

        def gather(j):
            b = lax.rem(j, RB)
            pltpu.async_copy(h_hbm.at[src_v.at[j]], rows_v.at[b], gsem.at[b])

        def wait_gather(j):
            b = lax.rem(j, RB)
            pltpu.make_async_copy(h_hbm.at[src_v.at[j]], rows_v.at[b],
                                  gsem.at[b]).wait()

        def scatter(j):
            b = lax.rem(j, RB)
            pltpu.async_copy(rows_v.at[b], agg_sh.at[dst_v.at[j]],
                             ssem.at[b], add=True)

        def wait_scatter(j):
            b = lax.rem(j, RB)
            pltpu.make_async_copy(rows_v.at[b], agg_sh.at[dst_v.at[j]],
                                  ssem.at[b]).wait()

        def prol(j, carry):
            gather(j)
            return carry

        lax.fori_loop(0, RB, prol, 0)

        def steady(j, carry):
            wait_gather(j)
            scatter(j)

            @pl.when(j >= 1)
            def _():
                wait_scatter(j - 1)

                @pl.when(j - 1 + RB < nb)
                def _():
                    gather(j - 1 + RB)

            return carry

        lax.fori_loop(0, nb, steady, 0)
        wait_scatter(nb - 1)

        plsc.subcore_barrier()
        pltpu.sync_copy(agg_sh.at[pl.ds(base, ROWS_T)],
                        out_hbm.at[c, pl.ds(base, ROWS_T)])

    return agg


_agg128 = _make_agg_kernel(128)
_agg64 = _make_agg_kernel(64)


def _norm(da, db):
    deg = da + db
    return jnp.where(deg > 0, lax.rsqrt(jnp.maximum(deg, 1.0)), 0.0)


def _mm1_body(x_ref, w_ref, doa_ref, dob_ref, o_ref):
    ns = _norm(doa_ref[...], dob_ref[...])
    o_ref[...] = jnp.dot(x_ref[...], w_ref[...],
                         preferred_element_type=jnp.float32) * ns


def _l2_body(p_ref, dia_ref, dib_ref, doa_ref, dob_ref,
             b1_ref, w2_ref, o_ref):
    nd = _norm(dia_ref[...], dib_ref[...])
    x = (p_ref[0] + p_ref[1]) * nd + b1_ref[...]
    x = jnp.maximum(x, 0.0)
    ns = _norm(doa_ref[...], dob_ref[...])
    o_ref[...] = jnp.dot(x, w2_ref[...],
                         preferred_element_type=jnp.float32) * ns


def _fin_body(p_ref, dia_ref, dib_ref, b2_ref, o_ref):
    nd = _norm(dia_ref[...], dib_ref[...])
    y = (p_ref[0] + p_ref[1]) * nd + b2_ref[...]
    m = jnp.max(y, axis=1, keepdims=True)
    e = jnp.exp(y - m)
    o_ref[...] = (y - m) - jnp.log(jnp.sum(e, axis=1, keepdims=True))


_RBK = 2560
_FBK = 2000


def _vec_spec(rb):
    return pl.BlockSpec((rb, 1), lambda i: (i, 0))


def _pair_spec(rb, d):
    return pl.BlockSpec((2, rb, d), lambda i: (0, i, 0))


def _full_spec(shape):
    return pl.BlockSpec(shape, lambda i: tuple(0 for _ in shape))


def _tc_mm1(x, w1, doa, dob):
    return pl.pallas_call(
        _mm1_body,
        grid=(NP // _RBK,),
        in_specs=[
            pl.BlockSpec((_RBK, 128), lambda i: (i, 0)),
            _full_spec((128, 128)),
            _vec_spec(_RBK), _vec_spec(_RBK),
        ],
        out_specs=pl.BlockSpec((_RBK, 128), lambda i: (i, 0)),
        out_shape=jax.ShapeDtypeStruct((NP, 128), jnp.float32),
    )(x, w1, doa, dob)


def _tc_l2(p, dia, dib, doa, dob, b1, w2):
    return pl.pallas_call(
        _l2_body,
        grid=(NP // _RBK,),
        in_specs=[
            _pair_spec(_RBK, 128),
            _vec_spec(_RBK), _vec_spec(_RBK), _vec_spec(_RBK), _vec_spec(_RBK),
            _full_spec((1, 128)),
            _full_spec((128, 64)),
        ],
        out_specs=pl.BlockSpec((_RBK, 64), lambda i: (i, 0)),
        out_shape=jax.ShapeDtypeStruct((NP, 64), jnp.float32),
    )(p, dia, dib, doa, dob, b1, w2)


def _tc_fin(p, dia, dib, b2):
    return pl.pallas_call(
        _fin_body,
        grid=(N // _FBK,),
        in_specs=[
            _pair_spec(_FBK, 64),
            _vec_spec(_FBK), _vec_spec(_FBK),
            _full_spec((1, 64)),
        ],
        out_specs=pl.BlockSpec((_FBK, 64), lambda i: (i, 0)),
        out_shape=jax.ShapeDtypeStruct((N, 64), jnp.float32),
    )(p, dia, dib, b2)


def kernel(features, edge_index, W1, b1, W2, b2):
    src = edge_index[0].astype(jnp.int32)
    dst = edge_index[1].astype(jnp.int32)
    pad = FBP * EB - E
    src_b = jnp.concatenate(
        [src, jnp.full((pad,), N, jnp.int32)]).reshape(FBP, EB)
    dst_b = jnp.concatenate(
        [dst, jnp.full((pad,), N, jnp.int32)]).reshape(FBP, EB)

    degp = _deg_kernel(src_b, dst_b)
    doa = degp[0, 0][:, None]
    dob = degp[1, 0][:, None]
    dia = degp[0, 1][:, None]
    dib = degp[1, 1][:, None]

    xpad = jnp.zeros((NP, 128), jnp.float32).at[:N].set(features)
    h1 = _tc_mm1(xpad, W1, doa, dob)
    p1 = _agg128(h1, src_b, dst_b)
    h2 = _tc_l2(p1, dia, dib, doa, dob,
                b1.reshape(1, 128), W2)
    p2 = _agg64(h2, src_b, dst_b)
    return _tc_fin(p2, dia, dib, b2.reshape(1, 64))


# speedup vs baseline: 1.9386x; 1.9029x over previous
import functools

import jax
import jax.numpy as jnp
from jax import lax
from jax.experimental import pallas as pl
from jax.experimental.pallas import tpu as pltpu
from jax.experimental.pallas import tpu_sc as plsc

N = 10000
NP = 10240
E = 320000
NC = 2
NS = 16
EB = 64
NW = NC * NS
NBLK = -(-E // (NW * EB))
EPAD = NW * NBLK * EB
ROWS_T = NP // NS
RB = 3
DEG_Q = 8

_MESH = plsc.VectorSubcoreMesh(core_axis_name="c", subcore_axis_name="s")
_SC_PARAMS = pltpu.CompilerParams(use_tc_tiling_on_sc=False)


@functools.partial(
    pl.kernel,
    out_type=jax.ShapeDtypeStruct((NC, 2, NP), jnp.float32),
    mesh=_MESH,
    scratch_types=[
        pltpu.VMEM((NBLK, EB), jnp.int32),
        pltpu.VMEM((NBLK, EB), jnp.int32),
        pltpu.VMEM((EB,), jnp.float32),
        pltpu.VMEM((ROWS_T,), jnp.float32),
        pltpu.VMEM_SHARED((NP,), jnp.float32),
        pltpu.VMEM_SHARED((NP,), jnp.float32),
        pltpu.SemaphoreType.DMA,
    ],
)
def _deg_kernel(src_hbm, dst_hbm, out_hbm,
                src_v, dst_v, ones_v, zbuf_v, ho_sh, hi_sh, sem):
    c = lax.axis_index("c")
    s = lax.axis_index("s")
    w = c * NS + s
    base = s * ROWS_T

    def zinit(i, carry):
        zbuf_v[pl.ds(i * 16, 16)] = jnp.zeros((16,), jnp.float32)
        return carry

    lax.fori_loop(0, ROWS_T // 16, zinit, 0)

    def oinit(i, carry):
        ones_v[pl.ds(i * 16, 16)] = jnp.ones((16,), jnp.float32)
        return carry

    lax.fori_loop(0, EB // 16, oinit, 0)

    pltpu.sync_copy(zbuf_v, ho_sh.at[pl.ds(base, ROWS_T)])
    pltpu.sync_copy(zbuf_v, hi_sh.at[pl.ds(base, ROWS_T)])
    pltpu.sync_copy(src_hbm.at[w], src_v)
    pltpu.sync_copy(dst_hbm.at[w], dst_v)
    plsc.subcore_barrier()

    def fire(t):
        j = lax.rem(t, NBLK)

        @pl.when(t < NBLK)
        def _():
            pltpu.async_copy(ones_v, ho_sh.at[src_v.at[j]], sem, add=True)

        @pl.when(t >= NBLK)
        def _():
            pltpu.async_copy(ones_v, hi_sh.at[dst_v.at[j]], sem, add=True)

    def drain_one():
        pltpu.make_async_copy(ones_v, ho_sh.at[src_v.at[0]], sem).wait()

    def prol(t, carry):
        fire(t)
        return carry

    lax.fori_loop(0, DEG_Q, prol, 0)

    def steady(t, carry):
        fire(t + DEG_Q)
        drain_one()
        return carry

    lax.fori_loop(0, 2 * NBLK - DEG_Q, steady, 0)

    def epi(t, carry):
        drain_one()
        return carry

    lax.fori_loop(0, DEG_Q, epi, 0)

    plsc.subcore_barrier()
    pltpu.sync_copy(ho_sh.at[pl.ds(base, ROWS_T)],
                    out_hbm.at[c, 0, pl.ds(base, ROWS_T)])
    pltpu.sync_copy(hi_sh.at[pl.ds(base, ROWS_T)],
                    out_hbm.at[c, 1, pl.ds(base, ROWS_T)])


def _make_agg_kernel(D):
    @functools.partial(
        pl.kernel,
        out_type=jax.ShapeDtypeStruct((NC, NP, D), jnp.float32),
        mesh=_MESH,
        scratch_types=[
            pltpu.VMEM((NBLK, EB), jnp.int32),
            pltpu.VMEM((NBLK, EB), jnp.int32),
            pltpu.VMEM((RB, EB, D), jnp.float32),
            pltpu.VMEM_SHARED((NP, D), jnp.float32),
            pltpu.SemaphoreType.DMA((RB,)),
            pltpu.SemaphoreType.DMA((RB,)),
        ],
        compiler_params=_SC_PARAMS,
    )
    def agg(h_hbm, src_hbm, dst_hbm, out_hbm,
            src_v, dst_v, rows_v, agg_sh, gsem, ssem):
        c = lax.axis_index("c")
        s = lax.axis_index("s")
        w = c * NS + s
        base = s * ROWS_T
        nb = NBLK

        def zinit(i, carry):
            rows_v[0, lax.div(i, D // 16), pl.ds(lax.rem(i, D // 16) * 16, 16)] = (
                jnp.zeros((16,), jnp.float32))
            return carry

        lax.fori_loop(0, EB * D // 16, zinit, 0)

        def zcopy(k, carry):
            pltpu.sync_copy(rows_v.at[0],
                            agg_sh.at[pl.ds(base + k * EB, EB)])
            return carry

        lax.fori_loop(0, ROWS_T // EB, zcopy, 0)

        pltpu.sync_copy(src_hbm.at[w], src_v)
        pltpu.sync_copy(dst_hbm.at[w], dst_v)
        plsc.subcore_barrier()

        def gather(j):
            b = lax.rem(j, RB)
            pltpu.async_copy(h_hbm.at[src_v.at[j]], rows_v.at[b], gsem.at[b])

        def wait_gather(j):
            b = lax.rem(j, RB)
            pltpu.make_async_copy(h_hbm.at[src_v.at[j]], rows_v.at[b],
                                  gsem.at[b]).wait()

        def scatter(j):
            b = lax.rem(j, RB)
            pltpu.async_copy(rows_v.at[b], agg_sh.at[dst_v.at[j]],
                             ssem.at[b], add=True)

        def wait_scatter(j):
            b = lax.rem(j, RB)
            pltpu.make_async_copy(rows_v.at[b], agg_sh.at[dst_v.at[j]],
                                  ssem.at[b]).wait()

        def prol(j, carry):
            gather(j)
            return carry

        lax.fori_loop(0, RB, prol, 0)

        def steady(j, carry):
            wait_gather(j)
            scatter(j)

            @pl.when(j >= 1)
            def _():
                wait_scatter(j - 1)

                @pl.when(j - 1 + RB < nb)
                def _():
                    gather(j - 1 + RB)

            return carry

        lax.fori_loop(0, nb, steady, 0)
        wait_scatter(nb - 1)

        plsc.subcore_barrier()
        pltpu.sync_copy(agg_sh.at[pl.ds(base, ROWS_T)],
                        out_hbm.at[c, pl.ds(base, ROWS_T)])

    return agg


_agg128 = _make_agg_kernel(128)
_agg64 = _make_agg_kernel(64)


def _norm(da, db):
    deg = da + db
    return jnp.where(deg > 0, lax.rsqrt(jnp.maximum(deg, 1.0)), 0.0)


def _mm1_body(x_ref, w_ref, doa_ref, dob_ref, o_ref):
    ns = _norm(doa_ref[...], dob_ref[...])
    o_ref[...] = jnp.dot(x_ref[...], w_ref[...],
                         preferred_element_type=jnp.float32) * ns


def _l2_body(p_ref, dia_ref, dib_ref, doa_ref, dob_ref,
             b1_ref, w2_ref, o_ref):
    nd = _norm(dia_ref[...], dib_ref[...])
    x = (p_ref[0] + p_ref[1]) * nd + b1_ref[...]
    x = jnp.maximum(x, 0.0)
    ns = _norm(doa_ref[...], dob_ref[...])
    o_ref[...] = jnp.dot(x, w2_ref[...],
                         preferred_element_type=jnp.float32) * ns


def _fin_body(p_ref, dia_ref, dib_ref, b2_ref, o_ref):
    nd = _norm(dia_ref[...], dib_ref[...])
    y = (p_ref[0] + p_ref[1]) * nd + b2_ref[...]
    m = jnp.max(y, axis=1, keepdims=True)
    e = jnp.exp(y - m)
    o_ref[...] = (y - m) - jnp.log(jnp.sum(e, axis=1, keepdims=True))


_RBK = 2560
_FBK = 2000


def _vec_spec(rb):
    return pl.BlockSpec((rb, 1), lambda i: (i, 0))


def _pair_spec(rb, d):
    return pl.BlockSpec((2, rb, d), lambda i: (0, i, 0))


def _full_spec(shape):
    return pl.BlockSpec(shape, lambda i: tuple(0 for _ in shape))


def _tc_mm1(x, w1, doa, dob):
    return pl.pallas_call(
        _mm1_body,
        grid=(NP // _RBK,),
        in_specs=[
            pl.BlockSpec((_RBK, 128), lambda i: (i, 0)),
            _full_spec((128, 128)),
            _vec_spec(_RBK), _vec_spec(_RBK),
        ],
        out_specs=pl.BlockSpec((_RBK, 128), lambda i: (i, 0)),
        out_shape=jax.ShapeDtypeStruct((NP, 128), jnp.float32),
    )(x, w1, doa, dob)


def _tc_l2(p, dia, dib, doa, dob, b1, w2):
    return pl.pallas_call(
        _l2_body,
        grid=(NP // _RBK,),
        in_specs=[
            _pair_spec(_RBK, 128),
            _vec_spec(_RBK), _vec_spec(_RBK), _vec_spec(_RBK), _vec_spec(_RBK),
            _full_spec((1, 128)),
            _full_spec((128, 64)),
        ],
        out_specs=pl.BlockSpec((_RBK, 64), lambda i: (i, 0)),
        out_shape=jax.ShapeDtypeStruct((NP, 64), jnp.float32),
    )(p, dia, dib, doa, dob, b1, w2)


def _tc_fin(p, dia, dib, b2):
    return pl.pallas_call(
        _fin_body,
        grid=(N // _FBK,),
        in_specs=[
            _pair_spec(_FBK, 64),
            _vec_spec(_FBK), _vec_spec(_FBK),
            _full_spec((1, 64)),
        ],
        out_specs=pl.BlockSpec((_FBK, 64), lambda i: (i, 0)),
        out_shape=jax.ShapeDtypeStruct((N, 64), jnp.float32),
    )(p, dia, dib, b2)


def kernel(features, edge_index, W1, b1, W2, b2):
    src = edge_index[0].astype(jnp.int32)
    dst = edge_index[1].astype(jnp.int32)
    pad = EPAD - E
    src_b = jnp.concatenate(
        [src, jnp.full((pad,), N, jnp.int32)]).reshape(NW, NBLK, EB)
    dst_b = jnp.concatenate(
        [dst, jnp.full((pad,), N, jnp.int32)]).reshape(NW, NBLK, EB)

    degp = _deg_kernel(src_b, dst_b)
    doa = degp[0, 0][:, None]
    dob = degp[1, 0][:, None]
    dia = degp[0, 1][:, None]
    dib = degp[1, 1][:, None]

    xpad = jnp.zeros((NP, 128), jnp.float32).at[:N].set(features)
    h1 = _tc_mm1(xpad, W1, doa, dob)
    p1 = _agg128(h1, src_b, dst_b)
    h2 = _tc_l2(p1, dia, dib, doa, dob,
                b1.reshape(1, 128), W2)
    p2 = _agg64(h2, src_b, dst_b)
    return _tc_fin(p2, dia, dib, b2.reshape(1, 64))
